# Initial kernel scaffold; baseline (speedup 1.0000x reference)
import jax
import jax.numpy as jnp
from jax.experimental import pallas as pl
from jax.experimental.pallas import tpu as pltpu
from jax.experimental.pallas import tpu_sc as plsc

NUM_FIELDS = 26
VOCAB = 100000
EMB = 32
BATCH = 16384
D_IN = NUM_FIELDS * EMB
H1, H2 = 256, 128
EPS = 1e-5

NIDX = BATCH * NUM_FIELDS
GW = 128


def _sc_gather(table2, table1, flat_idx):
    mesh = plsc.VectorSubcoreMesh(core_axis_name="c", subcore_axis_name="s")
    idx2d = flat_idx.reshape(1, NIDX)

    @pl.kernel(
        out_type=(
            jax.ShapeDtypeStruct((NIDX, EMB), jnp.float32),
            jax.ShapeDtypeStruct((NIDX, 1), jnp.float32),
        ),
        mesh=mesh,
    )
    def k(t2_hbm, t1_hbm, idx_hbm, o2_hbm, o1_hbm):
        def body(i_vmem, o2_vmem, o1_vmem):
            idx = i_vmem.at[0]
            pltpu.sync_copy(t2_hbm.at[idx], o2_vmem)
            pltpu.sync_copy(t1_hbm.at[idx], o1_vmem)

        pltpu.emit_pipeline(
            body,
            grid=(NIDX // GW,),
            in_specs=[pl.BlockSpec((1, GW), lambda i: (0, i))],
            out_specs=[
                pl.BlockSpec((GW, EMB), lambda i: (i, 0)),
                pl.BlockSpec((GW, 1), lambda i: (i, 0)),
            ],
            core_axis_name=("c", "s"),
            dimension_semantics=(pltpu.PARALLEL,),
        )(idx_hbm, o2_hbm, o1_hbm)

    return k(table2, table1, idx2d)


BB = 1024


def _tc_body(x_ref, fm1_ref, w1_ref, b1_ref, g1_ref, be1_ref, rm1_ref, rv1_ref,
             w2_ref, b2_ref, g2_ref, be2_ref, rm2_ref, rv2_ref, w3_ref, b3_ref,
             out_ref):
    x = x_ref[...]

    sum_e = x[:, 0:EMB]
    sum_sq = sum_e * sum_e
    for f in range(1, NUM_FIELDS):
        v = x[:, f * EMB:(f + 1) * EMB]
        sum_e = sum_e + v
        sum_sq = sum_sq + v * v
    fm2 = 0.5 * jnp.sum(sum_e * sum_e - sum_sq, axis=1, keepdims=True)

    fm1 = jnp.sum(fm1_ref[...], axis=1, keepdims=True)

    h = jnp.dot(x, w1_ref[...], preferred_element_type=jnp.float32) + b1_ref[...]
    h = (h - rm1_ref[...]) * (g1_ref[...] * jax.lax.rsqrt(rv1_ref[...] + EPS)) + be1_ref[...]
    h = jnp.maximum(h, 0.0)
    h = jnp.dot(h, w2_ref[...], preferred_element_type=jnp.float32) + b2_ref[...]
    h = (h - rm2_ref[...]) * (g2_ref[...] * jax.lax.rsqrt(rv2_ref[...] + EPS)) + be2_ref[...]
    h = jnp.maximum(h, 0.0)
    dnn = jnp.dot(h, w3_ref[...], preferred_element_type=jnp.float32) + b3_ref[...]

    out_ref[...] = jax.nn.sigmoid(fm1 + fm2 + dnn)


def _tc_head(x, fm1, W1, b1, g1, be1, rm1, rv1, W2, b2, g2, be2, rm2, rv2, W3, b3):
    grid = (BATCH // BB,)
    full = lambda shape: pl.BlockSpec(shape, lambda i: tuple(0 for _ in shape))
    return pl.pallas_call(
        _tc_body,
        grid=grid,
        in_specs=[
            pl.BlockSpec((BB, D_IN), lambda i: (i, 0)),
            pl.BlockSpec((BB, NUM_FIELDS), lambda i: (i, 0)),
            full((D_IN, H1)), full((1, H1)), full((1, H1)), full((1, H1)),
            full((1, H1)), full((1, H1)),
            full((H1, H2)), full((1, H2)), full((1, H2)), full((1, H2)),
            full((1, H2)), full((1, H2)),
            full((H2, 1)), full((1, 1)),
        ],
        out_specs=pl.BlockSpec((BB, 1), lambda i: (i, 0)),
        out_shape=jax.ShapeDtypeStruct((BATCH, 1), jnp.float32),
    )(x, fm1, W1, b1, g1, be1, rm1, rv1, W2, b2, g2, be2, rm2, rv2, W3, b3)


def kernel(X_sparse, emb1, emb2, W1, b1, g1, be1, rm1, rv1, W2, b2, g2, be2,
           rm2, rv2, W3, b3):
    offs = (jnp.arange(NUM_FIELDS, dtype=jnp.int32) * VOCAB)[None, :]
    flat_idx = (X_sparse.astype(jnp.int32) + offs).reshape(-1)
    t2 = emb2.reshape(NUM_FIELDS * VOCAB, EMB)
    t1 = emb1.reshape(NUM_FIELDS * VOCAB, 1)

    gath2, gath1 = _sc_gather(t2, t1, flat_idx)

    x = gath2.reshape(BATCH, D_IN)
    fm1 = gath1.reshape(BATCH, NUM_FIELDS)

    r = lambda a: a.reshape(1, -1)
    return _tc_head(x, fm1, W1, r(b1), r(g1), r(be1), r(rm1), r(rv1),
                    W2, r(b2), r(g2), r(be2), r(rm2), r(rv2), W3, r(b3))


# speedup vs baseline: 3.7090x; 3.7090x over previous
import jax
import jax.numpy as jnp
from jax import lax
from jax.experimental import pallas as pl
from jax.experimental.pallas import tpu as pltpu
from jax.experimental.pallas import tpu_sc as plsc

NUM_FIELDS = 26
VOCAB = 100000
EMB = 32
BATCH = 16384
D_IN = NUM_FIELDS * EMB
H1, H2 = 256, 128
EPS = 1e-5

IDX_CHUNK = 8192


def _sc_gather_t(t2T, t1T, xT_idx):
    mesh = plsc.VectorSubcoreMesh(core_axis_name="c", subcore_axis_name="s")

    @pl.kernel(
        out_type=(
            jax.ShapeDtypeStruct((D_IN, BATCH), jnp.float32),
            jax.ShapeDtypeStruct((NUM_FIELDS, BATCH), jnp.float32),
        ),
        mesh=mesh,
        scratch_types=[
            pltpu.VMEM((VOCAB,), jnp.float32),
            pltpu.VMEM((IDX_CHUNK,), jnp.int32),
            pltpu.VMEM((IDX_CHUNK,), jnp.float32),
        ],
        compiler_params=pltpu.CompilerParams(use_tc_tiling_on_sc=True,
                                             needs_layout_passes=False),
    )
    def k(t2T_hbm, t1T_hbm, idx_hbm, o2T_hbm, o1T_hbm, row_v, idx_v, out_v):
        w = lax.axis_index("s") * 2 + lax.axis_index("c")

        def gather_row(table_row_ref, f, out_row_ref):
            pltpu.sync_copy(table_row_ref, row_v)

            @pl.loop(0, BATCH, step=IDX_CHUNK)
            def _(c):
                pltpu.sync_copy(idx_hbm.at[f, pl.ds(c, IDX_CHUNK)], idx_v)

                @pl.loop(0, IDX_CHUNK, step=16)
                def _(j):
                    idx16 = idx_v[pl.ds(j, 16)]
                    out_v[pl.ds(j, 16)] = plsc.load_gather(row_v, [idx16])

                pltpu.sync_copy(out_v, out_row_ref.at[pl.ds(c, IDX_CHUNK)])

        @pl.loop(0, NUM_FIELDS)
        def _(f):
            r = f * EMB + w
            gather_row(t2T_hbm.at[r], f, o2T_hbm.at[r])

        @pl.when(w < NUM_FIELDS)
        def _():
            gather_row(t1T_hbm.at[w], w, o1T_hbm.at[w])

    return k(t2T, t1T, xT_idx)


BB = 2048


def _tc_body(xT_ref, fm1T_ref, w1_ref, b1_ref, g1_ref, be1_ref, rm1_ref,
             rv1_ref, w2_ref, b2_ref, g2_ref, be2_ref, rm2_ref, rv2_ref,
             w3_ref, b3_ref, out_ref):
    xT = xT_ref[...]

    sum_e = xT[0:EMB, :]
    sum_sq = sum_e * sum_e
    for f in range(1, NUM_FIELDS):
        v = xT[f * EMB:(f + 1) * EMB, :]
        sum_e = sum_e + v
        sum_sq = sum_sq + v * v
    fm2 = 0.5 * jnp.sum(sum_e * sum_e - sum_sq, axis=0, keepdims=True)

    fm1 = jnp.sum(fm1T_ref[...], axis=0, keepdims=True)

    dn = (((0,), (0,)), ((), ()))

    h = lax.dot_general(w1_ref[...], xT, dn,
                        preferred_element_type=jnp.float32) + b1_ref[...]
    h = (h - rm1_ref[...]) * (g1_ref[...] * lax.rsqrt(rv1_ref[...] + EPS)) + be1_ref[...]
    h = jnp.maximum(h, 0.0)
    h = lax.dot_general(w2_ref[...], h, dn,
                        preferred_element_type=jnp.float32) + b2_ref[...]
    h = (h - rm2_ref[...]) * (g2_ref[...] * lax.rsqrt(rv2_ref[...] + EPS)) + be2_ref[...]
    h = jnp.maximum(h, 0.0)
    dnn = lax.dot_general(w3_ref[...], h, dn,
                          preferred_element_type=jnp.float32) + b3_ref[...]

    out_ref[...] = jax.nn.sigmoid(fm1 + fm2 + dnn)


def _tc_head(xT, fm1T, W1, b1, g1, be1, rm1, rv1, W2, b2, g2, be2, rm2, rv2,
             W3, b3):
    grid = (BATCH // BB,)
    full = lambda shape: pl.BlockSpec(shape, lambda i: tuple(0 for _ in shape))
    return pl.pallas_call(
        _tc_body,
        grid=grid,
        in_specs=[
            pl.BlockSpec((D_IN, BB), lambda i: (0, i)),
            pl.BlockSpec((NUM_FIELDS, BB), lambda i: (0, i)),
            full((D_IN, H1)), full((H1, 1)), full((H1, 1)), full((H1, 1)),
            full((H1, 1)), full((H1, 1)),
            full((H1, H2)), full((H2, 1)), full((H2, 1)), full((H2, 1)),
            full((H2, 1)), full((H2, 1)),
            full((H2, 1)), full((1, 1)),
        ],
        out_specs=pl.BlockSpec((1, BB), lambda i: (0, i)),
        out_shape=jax.ShapeDtypeStruct((1, BATCH), jnp.float32),
    )(xT, fm1T, W1, b1, g1, be1, rm1, rv1, W2, b2, g2, be2, rm2, rv2, W3, b3)


def kernel(X_sparse, emb1, emb2, W1, b1, g1, be1, rm1, rv1, W2, b2, g2, be2,
           rm2, rv2, W3, b3):
    t2T = jnp.transpose(emb2, (0, 2, 1)).reshape(D_IN, VOCAB)
    t1T = jnp.transpose(emb1, (0, 2, 1)).reshape(NUM_FIELDS, VOCAB)
    xT_idx = jnp.transpose(X_sparse, (1, 0)).astype(jnp.int32)

    o2T, o1T = _sc_gather_t(t2T, t1T, xT_idx)

    r = lambda a: a.reshape(-1, 1)
    out_row = _tc_head(o2T, o1T, W1, r(b1), r(g1), r(be1), r(rm1), r(rv1),
                       W2, r(b2), r(g2), r(be2), r(rm2), r(rv2), r(W3), r(b3))
    return out_row.reshape(BATCH, 1)
